# baseline (speedup 1.0000x reference)
import functools

import jax
import jax.numpy as jnp
from jax import lax
from jax.experimental import pallas as pl
from jax.experimental.pallas import tpu as pltpu
from jax.experimental.pallas import tpu_sc as plsc

_C = 128
_NS = 16
_NBUF = 3
_LANES = 16


def _scale_rows(rv, ev, JB):
    @plsc.parallel_loop(0, _C // _LANES, unroll=2)
    def _scale(k):
        en16 = ev[pl.ds(k * _LANES, _LANES)]
        for l in range(_LANES):
            sc16 = jnp.take_along_axis(
                en16, jnp.full((_LANES,), l, jnp.int32), axis=0,
                mode="promise_in_bounds")
            e = k * _LANES + l
            for j in range(JB):
                sl = pl.ds(j * _LANES, _LANES)
                rv[e, sl] = rv[e, sl] * sc16


def _gc_body(NCH, RP, RPL, JB, x0, x1, si, ti, en, out0, out1,
             si3, ti3, en3, rows3, acc, *sems):
    isem = sems[0:_NBUF]
    gsem = sems[_NBUF:2 * _NBUF]
    ssem = sems[2 * _NBUF:3 * _NBUF]
    c = lax.axis_index("c")
    s = lax.axis_index("s")
    EPT = NCH * _C

    def fire_idx(g, b):
        base = pl.multiple_of(s * EPT + g * _C, _C)
        pltpu.async_copy(si.at[pl.ds(base, _C)], si3.at[b], isem[b])
        pltpu.async_copy(ti.at[pl.ds(base, _C)], ti3.at[b], isem[b])
        pltpu.async_copy(en.at[pl.ds(base, _C)], en3.at[b], isem[b])

    def wait_idx(g, b):
        base = pl.multiple_of(s * EPT + g * _C, _C)
        pltpu.make_async_copy(si.at[pl.ds(base, _C)], si3.at[b], isem[b]).wait()
        pltpu.make_async_copy(ti.at[pl.ds(base, _C)], ti3.at[b], isem[b]).wait()
        pltpu.make_async_copy(en.at[pl.ds(base, _C)], en3.at[b], isem[b]).wait()

    def fire_gather(b):
        @pl.when(c == 0)
        def _():
            pltpu.async_copy(x0.at[si3.at[b]], rows3.at[b], gsem[b])

        @pl.when(c == 1)
        def _():
            pltpu.async_copy(x1.at[si3.at[b]], rows3.at[b], gsem[b])

    def wait_gather(b):
        @pl.when(c == 0)
        def _():
            pltpu.make_async_copy(x0.at[si3.at[b]], rows3.at[b],
                                  gsem[b]).wait()

        @pl.when(c == 1)
        def _():
            pltpu.make_async_copy(x1.at[si3.at[b]], rows3.at[b],
                                  gsem[b]).wait()

    def fire_scatter(b):
        pltpu.async_copy(rows3.at[b], acc.at[ti3.at[b]], ssem[b], add=True)

    def wait_scatter(b):
        pltpu.make_async_copy(rows3.at[b], acc.at[ti3.at[b]], ssem[b]).wait()

    fire_idx(0, 0)

    r0 = rows3.at[0]

    @plsc.parallel_loop(0, _C, unroll=4)
    def _zrow(i):
        for j in range(JB):
            r0[i, pl.ds(j * _LANES, _LANES)] = jnp.zeros((_LANES,), jnp.float32)

    row0 = pl.multiple_of(s * RP, 8)

    def _zero_slice(nrows):
        full, rem = divmod(nrows, _C)
        for kblk in range(full):
            pltpu.sync_copy(r0, acc.at[pl.ds(row0 + kblk * _C, _C)])
        if rem:
            pltpu.sync_copy(r0.at[pl.ds(0, rem)],
                            acc.at[pl.ds(row0 + full * _C, rem)])

    @pl.when(s < _NS - 1)
    def _():
        _zero_slice(RP)

    @pl.when(s == _NS - 1)
    def _():
        _zero_slice(RPL)
    plsc.subcore_barrier()

    def _super(K, carry):
        for j in range(_NBUF):
            g = _NBUF * K + j
            b, nb, pv = j, (j + 1) % _NBUF, (j + 2) % _NBUF

            @pl.when(g < NCH)
            def _(g=g, b=b):
                wait_idx(g, b)
                fire_gather(b)

            @pl.when(jnp.logical_and(g - 2 >= 0, g - 2 < NCH))
            def _(b=nb):
                wait_scatter(b)

            @pl.when(g + 1 < NCH)
            def _(g=g, b=nb):
                fire_idx(g + 1, b)

            @pl.when(jnp.logical_and(g - 1 >= 0, g - 1 < NCH))
            def _(b=pv):
                wait_gather(b)
                _scale_rows(rows3.at[b], en3.at[b], JB)
                fire_scatter(b)
        return carry
    lax.fori_loop(0, NCH // _NBUF + 1, _super, 0)

    plsc.subcore_barrier()

    for last, nr in ((False, RP), (True, RPL)):
        @pl.when(jnp.logical_and(c == 0, (s == _NS - 1) == last))
        def _(nr=nr):
            pltpu.sync_copy(acc.at[pl.ds(row0, nr)], out0.at[pl.ds(row0, nr)])

        @pl.when(jnp.logical_and(c == 1, (s == _NS - 1) == last))
        def _(nr=nr):
            pltpu.sync_copy(acc.at[pl.ds(row0, nr)], out1.at[pl.ds(row0, nr)])


def kernel(x, eidx, enorm):
    N, D = x.shape
    E = eidx.shape[1]
    Dh = D // 2
    GRAIN = _NS * _C * _NBUF
    EPAD = -(-E // GRAIN) * GRAIN
    NCH = EPAD // (_NS * _C)
    RP = -(-(-(-N // _NS)) // 8) * 8
    RPL = N - (_NS - 1) * RP
    assert 0 < RPL <= RP

    si = jnp.pad(eidx[0].astype(jnp.int32), (0, EPAD - E))
    ti = jnp.pad(eidx[1].astype(jnp.int32), (0, EPAD - E))
    en = jnp.pad(enorm, (0, EPAD - E))
    x0 = x[:, :Dh]
    x1 = x[:, Dh:]

    mesh = plsc.VectorSubcoreMesh(core_axis_name="c", subcore_axis_name="s")
    out0, out1 = pl.kernel(
        functools.partial(_gc_body, NCH, RP, RPL, Dh // _LANES),
        out_type=(jax.ShapeDtypeStruct((N, Dh), jnp.float32),
                  jax.ShapeDtypeStruct((N, Dh), jnp.float32)),
        mesh=mesh,
        scratch_types=[
            pltpu.VMEM((_NBUF, _C), jnp.int32),
            pltpu.VMEM((_NBUF, _C), jnp.int32),
            pltpu.VMEM((_NBUF, _C), jnp.float32),
            pltpu.VMEM((_NBUF, _C, Dh), jnp.float32),
            pltpu.VMEM_SHARED((N, Dh), jnp.float32),
        ] + [pltpu.SemaphoreType.DMA] * (3 * _NBUF),
    )(x0, x1, si, ti, en)
    return jnp.concatenate([out0, out1], axis=1)


# speedup vs baseline: 2.1159x; 2.1159x over previous
import functools

import jax
import jax.numpy as jnp
from jax import lax
from jax.experimental import pallas as pl
from jax.experimental.pallas import tpu as pltpu
from jax.experimental.pallas import tpu_sc as plsc

_C = 112
_NS = 16
_NBUF = 3
_LANES = 16


def _scale_rows(rv, ev, JB):
    @plsc.parallel_loop(0, _C // _LANES, unroll=4)
    def _scale(k):
        en16 = ev[pl.ds(k * _LANES, _LANES)]
        for l in range(_LANES):
            sc16 = jnp.take_along_axis(
                en16, jnp.full((_LANES,), l, jnp.int32), axis=0,
                mode="promise_in_bounds")
            e = k * _LANES + l
            for j in range(JB):
                sl = pl.ds(j * _LANES, _LANES)
                rv[e, sl] = rv[e, sl] * sc16


def _gc_body(NCH, RP, JB, x0, x1, si, ti, en, out0, out1,
             si3, ti3, en3, rows3, acc, *sems):
    isem = sems[0:_NBUF]
    gsem = sems[_NBUF:2 * _NBUF]
    ssem = sems[2 * _NBUF:3 * _NBUF]
    c = lax.axis_index("c")
    s = lax.axis_index("s")
    EPT = NCH * _C

    def fire_idx(g, b):
        base = pl.multiple_of(s * EPT + g * _C, _C)
        pltpu.async_copy(si.at[pl.ds(base, _C)], si3.at[b], isem[b])
        pltpu.async_copy(ti.at[pl.ds(base, _C)], ti3.at[b], isem[b])
        pltpu.async_copy(en.at[pl.ds(base, _C)], en3.at[b], isem[b])

    def wait_idx(g, b):
        base = pl.multiple_of(s * EPT + g * _C, _C)
        pltpu.make_async_copy(si.at[pl.ds(base, _C)], si3.at[b], isem[b]).wait()
        pltpu.make_async_copy(ti.at[pl.ds(base, _C)], ti3.at[b], isem[b]).wait()
        pltpu.make_async_copy(en.at[pl.ds(base, _C)], en3.at[b], isem[b]).wait()

    def fire_gather(b):
        @pl.when(c == 0)
        def _():
            pltpu.async_copy(x0.at[si3.at[b]], rows3.at[b], gsem[b])

        @pl.when(c == 1)
        def _():
            pltpu.async_copy(x1.at[si3.at[b]], rows3.at[b], gsem[b])

    def wait_gather(b):
        @pl.when(c == 0)
        def _():
            pltpu.make_async_copy(x0.at[si3.at[b]], rows3.at[b],
                                  gsem[b]).wait()

        @pl.when(c == 1)
        def _():
            pltpu.make_async_copy(x1.at[si3.at[b]], rows3.at[b],
                                  gsem[b]).wait()

    def fire_scatter(b):
        pltpu.async_copy(rows3.at[b], acc.at[ti3.at[b]], ssem[b], add=True)

    def wait_scatter(b):
        pltpu.make_async_copy(rows3.at[b], acc.at[ti3.at[b]], ssem[b]).wait()

    fire_idx(0, 0)

    r0 = rows3.at[0]

    @plsc.parallel_loop(0, _C, unroll=4)
    def _zrow(i):
        for j in range(JB):
            r0[i, pl.ds(j * _LANES, _LANES)] = jnp.zeros((_LANES,), jnp.float32)

    row0 = pl.multiple_of(s * RP, 8)
    full, rem = divmod(RP, _C)
    for kblk in range(full):
        pltpu.sync_copy(r0, acc.at[pl.ds(row0 + kblk * _C, _C)])
    if rem:
        pltpu.sync_copy(r0.at[pl.ds(0, rem)],
                        acc.at[pl.ds(row0 + full * _C, rem)])
    plsc.subcore_barrier()

    def _super(K, carry):
        for j in range(_NBUF):
            g = _NBUF * K + j
            b, nb, pv = j, (j + 1) % _NBUF, (j + 2) % _NBUF

            @pl.when(g < NCH)
            def _(g=g, b=b):
                wait_idx(g, b)
                fire_gather(b)

            @pl.when(jnp.logical_and(g - 2 >= 0, g - 2 < NCH))
            def _(b=nb):
                wait_scatter(b)

            @pl.when(g + 1 < NCH)
            def _(g=g, b=nb):
                fire_idx(g + 1, b)

            @pl.when(jnp.logical_and(g - 1 >= 0, g - 1 < NCH))
            def _(b=pv):
                wait_gather(b)
                _scale_rows(rows3.at[b], en3.at[b], JB)
                fire_scatter(b)
        return carry
    lax.fori_loop(0, NCH // _NBUF + 1, _super, 0)

    plsc.subcore_barrier()

    @pl.when(c == 0)
    def _():
        pltpu.sync_copy(acc.at[pl.ds(row0, RP)], out0.at[pl.ds(row0, RP)])

    @pl.when(c == 1)
    def _():
        pltpu.sync_copy(acc.at[pl.ds(row0, RP)], out1.at[pl.ds(row0, RP)])


def kernel(x, eidx, enorm):
    N, D = x.shape
    E = eidx.shape[1]
    Dh = D // 2
    GRAIN = _NS * _C * _NBUF
    EPAD = -(-E // GRAIN) * GRAIN
    NCH = EPAD // (_NS * _C)
    NP = -(-N // (_NS * 8)) * (_NS * 8)
    RP = NP // _NS

    si = jnp.pad(eidx[0].astype(jnp.int32), (0, EPAD - E))
    ti = jnp.pad(eidx[1].astype(jnp.int32), (0, EPAD - E))
    en = jnp.pad(enorm, (0, EPAD - E))
    x0 = x[:, :Dh]
    x1 = x[:, Dh:]

    mesh = plsc.VectorSubcoreMesh(core_axis_name="c", subcore_axis_name="s")
    out0, out1 = pl.kernel(
        functools.partial(_gc_body, NCH, RP, Dh // _LANES),
        out_type=(jax.ShapeDtypeStruct((NP, Dh), jnp.float32),
                  jax.ShapeDtypeStruct((NP, Dh), jnp.float32)),
        mesh=mesh,
        scratch_types=[
            pltpu.VMEM((_NBUF, _C), jnp.int32),
            pltpu.VMEM((_NBUF, _C), jnp.int32),
            pltpu.VMEM((_NBUF, _C), jnp.float32),
            pltpu.VMEM((_NBUF, _C, Dh), jnp.float32),
            pltpu.VMEM_SHARED((NP, Dh), jnp.float32),
        ] + [pltpu.SemaphoreType.DMA] * (3 * _NBUF),
    )(x0, x1, si, ti, en)
    return jnp.concatenate([out0[:N], out1[:N]], axis=1)


# speedup vs baseline: 2.3174x; 1.0952x over previous
import functools

import jax
import jax.numpy as jnp
from jax import lax
from jax.experimental import pallas as pl
from jax.experimental.pallas import tpu as pltpu
from jax.experimental.pallas import tpu_sc as plsc

_C = 112
_NS = 16
_NBUF = 3
_LANES = 16


def _scale_rows(rv, ev, JB):
    @plsc.parallel_loop(0, _C // _LANES, unroll=2)
    def _scale(k):
        en16 = ev[pl.ds(k * _LANES, _LANES)]
        for l in range(_LANES):
            sc16 = jnp.take_along_axis(
                en16, jnp.full((_LANES,), l, jnp.int32), axis=0,
                mode="promise_in_bounds")
            e = k * _LANES + l
            for j in range(JB):
                sl = pl.ds(j * _LANES, _LANES)
                rv[e, sl] = rv[e, sl] * sc16


def _gc_body(NCH, RP, JB, x0, x1, si, ti, en, out0, out1,
             si3, ti3, en3, rows3, acc, *sems):
    isem = sems[0:_NBUF]
    gsem = sems[_NBUF:2 * _NBUF]
    ssem = sems[2 * _NBUF:3 * _NBUF]
    c = lax.axis_index("c")
    s = lax.axis_index("s")
    EPT = NCH * _C

    def fire_idx(g, b):
        base = pl.multiple_of(s * EPT + g * _C, _C)
        pltpu.async_copy(si.at[pl.ds(base, _C)], si3.at[b], isem[b])
        pltpu.async_copy(ti.at[pl.ds(base, _C)], ti3.at[b], isem[b])
        pltpu.async_copy(en.at[pl.ds(base, _C)], en3.at[b], isem[b])

    def wait_idx(g, b):
        base = pl.multiple_of(s * EPT + g * _C, _C)
        pltpu.make_async_copy(si.at[pl.ds(base, _C)], si3.at[b], isem[b]).wait()
        pltpu.make_async_copy(ti.at[pl.ds(base, _C)], ti3.at[b], isem[b]).wait()
        pltpu.make_async_copy(en.at[pl.ds(base, _C)], en3.at[b], isem[b]).wait()

    def fire_gather(b):
        @pl.when(c == 0)
        def _():
            pltpu.async_copy(x0.at[si3.at[b]], rows3.at[b], gsem[b])

        @pl.when(c == 1)
        def _():
            pltpu.async_copy(x1.at[si3.at[b]], rows3.at[b], gsem[b])

    def wait_gather(b):
        @pl.when(c == 0)
        def _():
            pltpu.make_async_copy(x0.at[si3.at[b]], rows3.at[b],
                                  gsem[b]).wait()

        @pl.when(c == 1)
        def _():
            pltpu.make_async_copy(x1.at[si3.at[b]], rows3.at[b],
                                  gsem[b]).wait()

    def fire_scatter(b):
        pltpu.async_copy(rows3.at[b], acc.at[ti3.at[b]], ssem[b], add=True)

    def wait_scatter(b):
        pltpu.make_async_copy(rows3.at[b], acc.at[ti3.at[b]], ssem[b]).wait()

    fire_idx(0, 0)

    r0 = rows3.at[0]

    @plsc.parallel_loop(0, _C, unroll=4)
    def _zrow(i):
        for j in range(JB):
            r0[i, pl.ds(j * _LANES, _LANES)] = jnp.zeros((_LANES,), jnp.float32)

    row0 = pl.multiple_of(s * RP, 8)
    full, rem = divmod(RP, _C)
    for kblk in range(full):
        pltpu.sync_copy(r0, acc.at[pl.ds(row0 + kblk * _C, _C)])
    if rem:
        pltpu.sync_copy(r0.at[pl.ds(0, rem)],
                        acc.at[pl.ds(row0 + full * _C, rem)])
    plsc.subcore_barrier()

    def _super(K, carry):
        for j in range(_NBUF):
            g = _NBUF * K + j
            b, nb, pv = j, (j + 1) % _NBUF, (j + 2) % _NBUF

            @pl.when(g < NCH)
            def _(g=g, b=b):
                wait_idx(g, b)
                fire_gather(b)

            @pl.when(jnp.logical_and(g - 2 >= 0, g - 2 < NCH))
            def _(b=nb):
                wait_scatter(b)

            @pl.when(g + 1 < NCH)
            def _(g=g, b=nb):
                fire_idx(g + 1, b)

            @pl.when(jnp.logical_and(g - 1 >= 0, g - 1 < NCH))
            def _(b=pv):
                wait_gather(b)
                _scale_rows(rows3.at[b], en3.at[b], JB)
                fire_scatter(b)
        return carry
    lax.fori_loop(0, NCH // _NBUF + 1, _super, 0)

    plsc.subcore_barrier()

    @pl.when(c == 0)
    def _():
        pltpu.sync_copy(acc.at[pl.ds(row0, RP)], out0.at[pl.ds(row0, RP)])

    @pl.when(c == 1)
    def _():
        pltpu.sync_copy(acc.at[pl.ds(row0, RP)], out1.at[pl.ds(row0, RP)])


def kernel(x, eidx, enorm):
    N, D = x.shape
    E = eidx.shape[1]
    Dh = D // 2
    GRAIN = _NS * _C * _NBUF
    EPAD = -(-E // GRAIN) * GRAIN
    NCH = EPAD // (_NS * _C)
    NP = -(-N // (_NS * 8)) * (_NS * 8)
    RP = NP // _NS

    si = jnp.pad(eidx[0].astype(jnp.int32), (0, EPAD - E))
    ti = jnp.pad(eidx[1].astype(jnp.int32), (0, EPAD - E))
    en = jnp.pad(enorm, (0, EPAD - E))
    x0 = x[:, :Dh]
    x1 = x[:, Dh:]

    mesh = plsc.VectorSubcoreMesh(core_axis_name="c", subcore_axis_name="s")
    out0, out1 = pl.kernel(
        functools.partial(_gc_body, NCH, RP, Dh // _LANES),
        out_type=(jax.ShapeDtypeStruct((NP, Dh), jnp.float32),
                  jax.ShapeDtypeStruct((NP, Dh), jnp.float32)),
        mesh=mesh,
        scratch_types=[
            pltpu.VMEM((_NBUF, _C), jnp.int32),
            pltpu.VMEM((_NBUF, _C), jnp.int32),
            pltpu.VMEM((_NBUF, _C), jnp.float32),
            pltpu.VMEM((_NBUF, _C, Dh), jnp.float32),
            pltpu.VMEM_SHARED((NP, Dh), jnp.float32),
        ] + [pltpu.SemaphoreType.DMA] * (3 * _NBUF),
    )(x0, x1, si, ti, en)
    return jnp.concatenate([out0[:N], out1[:N]], axis=1)
